# baseline (speedup 1.0000x reference)
import functools

import jax
import jax.numpy as jnp
from jax import lax
from jax.experimental import pallas as pl
from jax.experimental.pallas import tpu as pltpu

_EPS = 1e-5
_NEG = -3.0e38

_PARAMS_1D = pltpu.CompilerParams(
    dimension_semantics=("parallel",),
    vmem_limit_bytes=64 * 1024 * 1024,
)


def _full_spec(shape):
    nd = len(shape)
    return pl.BlockSpec(shape, lambda *_: (0,) * nd)


def _make_col_mask(rc_ref, *, W):
    cols = rc_ref[1:2, :]
    cache = {}

    def col_mask(dx):
        if dx == 0:
            return None
        if dx not in cache:
            cache[dx] = (cols >= -dx) if dx < 0 else (cols < W - dx)
        return cache[dx]

    return col_mask


def _dw_conv(scr, wdw_ref, K, dil, *, pad, W, HW, col_mask):
    half = (K // 2) * dil
    acc = None
    for kw in range(K):
        dx = kw * dil - half
        inner = None
        for kh in range(K):
            dy = kh * dil - half
            off = pad + dy * W + dx
            t = scr[:, off:off + HW] * wdw_ref[:, kh * K + kw:kh * K + kw + 1]
            inner = t if inner is None else inner + t
        m = col_mask(dx)
        if m is not None:
            inner = jnp.where(m, inner, 0.0)
        acc = inner if acc is None else acc + inner
    return acc


def _max_pool(mscr, *, pad, W, HW, col_mask):
    acc = None
    for dx in (-1, 0, 1):
        inner = None
        for dy in (-1, 0, 1):
            off = pad + dy * W + dx
            v = mscr[:, off:off + HW]
            inner = v if inner is None else jnp.maximum(inner, v)
        m = col_mask(dx)
        if m is not None:
            inner = jnp.where(m, inner, _NEG)
        acc = inner if acc is None else jnp.maximum(acc, inner)
    return acc


def _avg_pool(rscr, inv_cnt, *, pad, W, HW, col_mask):
    acc = None
    for dx in (-1, 0, 1):
        inner = None
        for dy in (-1, 0, 1):
            off = pad + dy * W + dx
            v = rscr[:, off:off + HW]
            inner = v if inner is None else inner + v
        m = col_mask(dx)
        if m is not None:
            inner = jnp.where(m, inner, 0.0)
        acc = inner if acc is None else acc + inner
    return acc * inv_cnt


def _stats(stats_ref, j, y):
    stats_ref[0, 2 * j] = jnp.sum(y, axis=1, keepdims=True)
    stats_ref[0, 2 * j + 1] = jnp.sum(y * y, axis=1, keepdims=True)


def _p1_kernel(rc_ref, inv_ref, x_ref,
               wdw_s3, wpw_s3, wdw_s5, wpw_s5,
               wdw_d3, wpw_d3, wdw_d5, wpw_d5,
               o_s3a, o_s5a, stats_ref,
               relu_scr, raw_scr, max_scr,
               *, C, H, W, pad):
    HW = H * W
    col_mask = _make_col_mask(rc_ref, W=W)
    geo = dict(pad=pad, W=W, HW=HW, col_mask=col_mask)

    x = x_ref[0]
    zpad = jnp.zeros((C, pad), jnp.float32)
    relu_scr[:, :pad] = zpad
    relu_scr[:, pad + HW:] = zpad
    relu_scr[:, pad:pad + HW] = jnp.maximum(x, 0.0)
    raw_scr[:, :pad] = zpad
    raw_scr[:, pad + HW:] = zpad
    raw_scr[:, pad:pad + HW] = x
    npad = jnp.full((C, pad), _NEG, jnp.float32)
    max_scr[:, :pad] = npad
    max_scr[:, pad + HW:] = npad
    max_scr[:, pad:pad + HW] = x

    for j, (o_ref, wdw, wpw, K, dil) in enumerate(
            ((o_s3a, wdw_s3, wpw_s3, 3, 1),
             (o_s5a, wdw_s5, wpw_s5, 5, 1),
             (None, wdw_d3, wpw_d3, 3, 2),
             (None, wdw_d5, wpw_d5, 5, 2))):
        dw = _dw_conv(relu_scr, wdw, K, dil, **geo)
        y = jnp.dot(wpw[...], dw, preferred_element_type=jnp.float32)
        _stats(stats_ref, j, y)
        if o_ref is not None:
            o_ref[0] = y.astype(jnp.bfloat16)

    _stats(stats_ref, 4, _max_pool(max_scr, **geo))
    _stats(stats_ref, 5, _avg_pool(raw_scr, inv_ref[...], **geo))


def _pass1(xf, rc, inv_cnt, weights, *, N, C, H, W, pad):
    HW = H * W
    img_spec = pl.BlockSpec((1, C, HW), lambda n: (n, 0, 0))
    return pl.pallas_call(
        functools.partial(_p1_kernel, C=C, H=H, W=W, pad=pad),
        grid=(N,),
        in_specs=[_full_spec(rc.shape), _full_spec(inv_cnt.shape), img_spec]
        + [_full_spec(w.shape) for w in weights],
        out_specs=[img_spec, img_spec,
                   pl.BlockSpec((1, 12, C, 1), lambda n: (n, 0, 0, 0))],
        out_shape=[jax.ShapeDtypeStruct((N, C, HW), jnp.bfloat16)] * 2
        + [jax.ShapeDtypeStruct((N, 12, C, 1), jnp.float32)],
        scratch_shapes=[pltpu.VMEM((C, HW + 2 * pad), jnp.float32)] * 3,
        compiler_params=_PARAMS_1D,
    )(rc, inv_cnt, xf, *weights)


def _p2_kernel(rc_ref, y3_ref, y5_ref,
               wdw2_s3, wpw2_s3, wdw2_s5, wpw2_s5, bn_ref,
               o_s3, o_s5, stats_ref, scr,
               *, C, H, W, pad):
    HW = H * W
    col_mask = _make_col_mask(rc_ref, W=W)
    geo = dict(pad=pad, W=W, HW=HW, col_mask=col_mask)

    zpad = jnp.zeros((C, pad), jnp.float32)
    scr[:, :pad] = zpad
    scr[:, pad + HW:] = zpad
    bn = bn_ref[...]

    for j, (y_ref, wdw, wpw, K, o_ref) in enumerate(
            ((y3_ref, wdw2_s3, wpw2_s3, 3, o_s3),
             (y5_ref, wdw2_s5, wpw2_s5, 5, o_s5))):
        y = y_ref[0].astype(jnp.float32)
        scr[:, pad:pad + HW] = jnp.maximum(
            y * bn[:, 2 * j:2 * j + 1] + bn[:, 2 * j + 1:2 * j + 2], 0.0)
        dw = _dw_conv(scr, wdw, K, 1, **geo)
        out = jnp.dot(wpw[...], dw, preferred_element_type=jnp.float32)
        _stats(stats_ref, j, out)
        o_ref[0] = out.astype(jnp.bfloat16)


def _pass2(y3, y5, rc, weights, *, N, C, H, W, pad):
    HW = H * W
    img_spec = pl.BlockSpec((1, C, HW), lambda n: (n, 0, 0))
    return pl.pallas_call(
        functools.partial(_p2_kernel, C=C, H=H, W=W, pad=pad),
        grid=(N,),
        in_specs=[_full_spec(rc.shape), img_spec, img_spec]
        + [_full_spec(w.shape) for w in weights],
        out_specs=[img_spec, img_spec,
                   pl.BlockSpec((1, 4, C, 1), lambda n: (n, 0, 0, 0))],
        out_shape=[jax.ShapeDtypeStruct((N, C, HW), jnp.bfloat16)] * 2
        + [jax.ShapeDtypeStruct((N, 4, C, 1), jnp.float32)],
        scratch_shapes=[pltpu.VMEM((C, HW + 2 * pad), jnp.float32)],
        compiler_params=_PARAMS_1D,
    )(rc, y3, y5, *weights)


def _p3_kernel(rc_ref, inv_ref, x_ref, s3_ref, s5_ref,
               wdw_d3, wpw_d3, wdw_d5, wpw_d5, sb_ref,
               o_ref, relu_scr, raw_scr, max_scr,
               *, C, H, W, pad):
    HW = H * W
    col_mask = _make_col_mask(rc_ref, W=W)
    geo = dict(pad=pad, W=W, HW=HW, col_mask=col_mask)

    x = x_ref[0]
    zpad = jnp.zeros((C, pad), jnp.float32)
    relu_scr[:, :pad] = zpad
    relu_scr[:, pad + HW:] = zpad
    relu_scr[:, pad:pad + HW] = jnp.maximum(x, 0.0)
    raw_scr[:, :pad] = zpad
    raw_scr[:, pad + HW:] = zpad
    raw_scr[:, pad:pad + HW] = x
    npad = jnp.full((C, pad), _NEG, jnp.float32)
    max_scr[:, :pad] = npad
    max_scr[:, pad + HW:] = npad
    max_scr[:, pad:pad + HW] = x

    sb = sb_ref[...]
    acc = x * sb[:, 4:5] + sb[:, 5:6]
    acc = acc + s3_ref[0].astype(jnp.float32) * sb[:, 0:1]
    acc = acc + s5_ref[0].astype(jnp.float32) * sb[:, 1:2]
    acc = acc + _max_pool(max_scr, **geo) * sb[:, 2:3]
    acc = acc + _avg_pool(raw_scr, inv_ref[...], **geo) * sb[:, 3:4]
    dw = _dw_conv(relu_scr, wdw_d3, 3, 2, **geo)
    acc = acc + jnp.dot(wpw_d3[...], dw, preferred_element_type=jnp.float32)
    dw = _dw_conv(relu_scr, wdw_d5, 5, 2, **geo)
    acc = acc + jnp.dot(wpw_d5[...], dw, preferred_element_type=jnp.float32)
    o_ref[0] = acc


def _pass3(xf, s3, s5, rc, inv_cnt, weights, *, N, C, H, W, pad):
    HW = H * W
    img_spec = pl.BlockSpec((1, C, HW), lambda n: (n, 0, 0))
    return pl.pallas_call(
        functools.partial(_p3_kernel, C=C, H=H, W=W, pad=pad),
        grid=(N,),
        in_specs=[_full_spec(rc.shape), _full_spec(inv_cnt.shape),
                  img_spec, img_spec, img_spec]
        + [_full_spec(w.shape) for w in weights],
        out_specs=img_spec,
        out_shape=jax.ShapeDtypeStruct((N, C, HW), jnp.float32),
        scratch_shapes=[pltpu.VMEM((C, HW + 2 * pad), jnp.float32)] * 3,
        compiler_params=_PARAMS_1D,
    )(rc, inv_cnt, xf, s3, s5, *weights)


def kernel(x, sep3_dw1, sep3_pw1, sep3_dw2, sep3_pw2,
           sep5_dw1, sep5_pw1, sep5_dw2, sep5_pw2,
           dil3_dw, dil3_pw, dil5_dw, dil5_pw, alphas):
    N, C, H, W = x.shape
    HW = H * W
    f32 = jnp.float32
    x = x.astype(f32)
    xf = x.reshape(N, C, HW)
    alphas = jnp.asarray(alphas, f32)

    pad = ((4 * W + 4 + 127) // 128) * 128

    idx = jnp.arange(HW, dtype=jnp.int32)
    rows = idx // W
    cols = idx % W
    rc = jnp.stack([rows, cols]).astype(jnp.int32)
    cnt = ((jnp.minimum(rows, 1) + jnp.minimum(H - 1 - rows, 1) + 1)
           * (jnp.minimum(cols, 1) + jnp.minimum(W - 1 - cols, 1) + 1))
    inv_cnt = (1.0 / cnt.astype(f32)).reshape(1, HW)

    dwW = lambda a: a.reshape(C, -1).astype(f32)
    pwW = lambda a: a[:, :, 0, 0].astype(f32)

    w1 = (dwW(sep3_dw1), pwW(sep3_pw1), dwW(sep5_dw1), pwW(sep5_pw1),
          dwW(dil3_dw), pwW(dil3_pw), dwW(dil5_dw), pwW(dil5_pw))
    y_s3a, y_s5a, stats1 = _pass1(xf, rc, inv_cnt, w1,
                                  N=N, C=C, H=H, W=W, pad=pad)

    total = jnp.float32(N * HW)
    st1 = jnp.sum(stats1[..., 0], axis=0)

    def finalize(st, j):
        s, ss = st[2 * j], st[2 * j + 1]
        m = s / total
        v = jnp.maximum(ss / total - m * m, 0.0)
        sc = lax.rsqrt(v + _EPS)
        return sc, -m * sc

    sc_s3a, bi_s3a = finalize(st1, 0)
    sc_s5a, bi_s5a = finalize(st1, 1)
    bn_mid = jnp.stack([sc_s3a, bi_s3a, sc_s5a, bi_s5a], axis=1)

    w2 = (dwW(sep3_dw2), pwW(sep3_pw2), dwW(sep5_dw2), pwW(sep5_pw2), bn_mid)
    y_s3, y_s5, stats2 = _pass2(y_s3a, y_s5a, rc, w2,
                                N=N, C=C, H=H, W=W, pad=pad)
    st2 = jnp.sum(stats2[..., 0], axis=0)

    sc_d3, bi_d3 = finalize(st1, 2)
    sc_d5, bi_d5 = finalize(st1, 3)
    sc_mx, bi_mx = finalize(st1, 4)
    sc_av, bi_av = finalize(st1, 5)
    sc_s3, bi_s3 = finalize(st2, 0)
    sc_s5, bi_s5 = finalize(st2, 1)

    total_bias = (alphas[0] * bi_mx + alphas[1] * bi_av
                  + alphas[3] * bi_s3 + alphas[4] * bi_s5
                  + alphas[5] * bi_d3 + alphas[6] * bi_d5)
    zeros = jnp.zeros((C,), f32)
    sb = jnp.stack([alphas[3] * sc_s3, alphas[4] * sc_s5,
                    alphas[0] * sc_mx, alphas[1] * sc_av,
                    jnp.full((C,), alphas[2], f32), total_bias,
                    zeros, zeros], axis=1)

    wpw_d3f = (alphas[5] * sc_d3)[:, None] * pwW(dil3_pw)
    wpw_d5f = (alphas[6] * sc_d5)[:, None] * pwW(dil5_pw)
    w3 = (dwW(dil3_dw), wpw_d3f, dwW(dil5_dw), wpw_d5f, sb)

    out = _pass3(xf, y_s3, y_s5, rc, inv_cnt, w3,
                 N=N, C=C, H=H, W=W, pad=pad)
    return out.reshape(N, C, H, W)


# speedup vs baseline: 1.4066x; 1.4066x over previous
import functools

import jax
import jax.numpy as jnp
from jax import lax
from jax.experimental import pallas as pl
from jax.experimental.pallas import tpu as pltpu

_EPS = 1e-5
_NEG = -3.0e38

_PARAMS_1D = pltpu.CompilerParams(
    dimension_semantics=("parallel",),
    vmem_limit_bytes=64 * 1024 * 1024,
)


def _full_spec(shape):
    nd = len(shape)
    return pl.BlockSpec(shape, lambda *_: (0,) * nd)


def _make_col_mask(rc_ref, *, W):
    cols = rc_ref[1:2, :]
    cache = {}

    def col_mask(dx):
        if dx == 0:
            return None
        if dx not in cache:
            cache[dx] = (cols >= -dx) if dx < 0 else (cols < W - dx)
        return cache[dx]

    return col_mask


def _make_slices(scr, *, pad, W, HW):
    cache = {}

    def get(dy, dx):
        key = (dy, dx)
        if key not in cache:
            off = pad + dy * W + dx
            cache[key] = scr[:, off:off + HW]
        return cache[key]

    return get


def _dw_conv(get, wdw_ref, K, dil, col_mask):
    half = (K // 2) * dil
    acc = None
    for kw in range(K):
        dx = kw * dil - half
        inner = None
        for kh in range(K):
            dy = kh * dil - half
            t = get(dy, dx) * wdw_ref[:, kh * K + kw:kh * K + kw + 1]
            inner = t if inner is None else inner + t
        m = col_mask(dx)
        if m is not None:
            inner = jnp.where(m, inner, 0.0)
        acc = inner if acc is None else acc + inner
    return acc


def _max_pool(get, col_mask):
    acc = None
    for dx in (-1, 0, 1):
        inner = None
        for dy in (-1, 0, 1):
            v = get(dy, dx)
            inner = v if inner is None else jnp.maximum(inner, v)
        m = col_mask(dx)
        if m is not None:
            inner = jnp.where(m, inner, _NEG)
        acc = inner if acc is None else jnp.maximum(acc, inner)
    return acc


def _avg_pool(get, inv_cnt, col_mask):
    acc = None
    for dx in (-1, 0, 1):
        inner = None
        for dy in (-1, 0, 1):
            v = get(dy, dx)
            inner = v if inner is None else inner + v
        m = col_mask(dx)
        if m is not None:
            inner = jnp.where(m, inner, 0.0)
        acc = inner if acc is None else acc + inner
    return acc * inv_cnt


def _fill(scr, vals, border, *, BC, pad, HW):
    scr[:, :pad] = jnp.full((BC, pad), border, jnp.float32)
    scr[:, pad + HW:] = jnp.full((BC, pad), border, jnp.float32)
    scr[:, pad:pad + HW] = vals


def _p1_kernel(rc_ref, inv_ref, x_ref,
               wdw_s3, wpw_s3, wdw_s5, wpw_s5,
               wdw_d3, wpw_d3, wdw_d5, wpw_d5,
               o_s3a, o_s5a, stats_ref,
               relu_scr, raw_scr, max_scr,
               *, B, C, H, W, pad):
    HW = H * W
    BC = B * C
    col_mask = _make_col_mask(rc_ref, W=W)
    geo = dict(pad=pad, W=W, HW=HW)

    x = x_ref[...].reshape(BC, HW)
    _fill(relu_scr, jnp.maximum(x, 0.0), 0.0, BC=BC, pad=pad, HW=HW)
    _fill(raw_scr, x, 0.0, BC=BC, pad=pad, HW=HW)
    _fill(max_scr, x, _NEG, BC=BC, pad=pad, HW=HW)
    get_r = _make_slices(relu_scr, **geo)

    for j, (o_ref, wdw, wpw, K, dil) in enumerate(
            ((o_s3a, wdw_s3, wpw_s3, 3, 1),
             (o_s5a, wdw_s5, wpw_s5, 5, 1),
             (None, wdw_d3, wpw_d3, 3, 2),
             (None, wdw_d5, wpw_d5, 5, 2))):
        dw = _dw_conv(get_r, wdw, K, dil, col_mask)
        for b in range(B):
            y = jnp.dot(wpw[...], dw[b * C:(b + 1) * C],
                        preferred_element_type=jnp.float32)
            stats_ref[b, 2 * j] = jnp.sum(y, axis=1, keepdims=True)
            stats_ref[b, 2 * j + 1] = jnp.sum(y * y, axis=1, keepdims=True)
            if o_ref is not None:
                o_ref[b] = y.astype(jnp.bfloat16)

    mx = _max_pool(_make_slices(max_scr, **geo), col_mask)
    av = _avg_pool(_make_slices(raw_scr, **geo), inv_ref[...], col_mask)
    for j, y in ((4, mx), (5, av)):
        for b in range(B):
            yb = y[b * C:(b + 1) * C]
            stats_ref[b, 2 * j] = jnp.sum(yb, axis=1, keepdims=True)
            stats_ref[b, 2 * j + 1] = jnp.sum(yb * yb, axis=1, keepdims=True)


def _pass1(xf, rc, inv_cnt, weights, *, B, N, C, H, W, pad):
    HW = H * W
    G = N // B
    img_spec = pl.BlockSpec((B, C, HW), lambda n: (n, 0, 0))
    return pl.pallas_call(
        functools.partial(_p1_kernel, B=B, C=C, H=H, W=W, pad=pad),
        grid=(G,),
        in_specs=[_full_spec(rc.shape), _full_spec(inv_cnt.shape), img_spec]
        + [_full_spec(w.shape) for w in weights],
        out_specs=[img_spec, img_spec,
                   pl.BlockSpec((B, 12, C, 1), lambda n: (n, 0, 0, 0))],
        out_shape=[jax.ShapeDtypeStruct((N, C, HW), jnp.bfloat16)] * 2
        + [jax.ShapeDtypeStruct((N, 12, C, 1), jnp.float32)],
        scratch_shapes=[pltpu.VMEM((B * C, HW + 2 * pad), jnp.float32)] * 3,
        compiler_params=_PARAMS_1D,
    )(rc, inv_cnt, xf, *weights)


def _p2_kernel(rc_ref, y3_ref, y5_ref,
               wdw2_s3, wpw2_s3, wdw2_s5, wpw2_s5, bn_ref,
               o_s3, o_s5, stats_ref, scr,
               *, B, C, H, W, pad):
    HW = H * W
    BC = B * C
    col_mask = _make_col_mask(rc_ref, W=W)
    bn = bn_ref[...]

    for j, (y_ref, wdw, wpw, K, o_ref) in enumerate(
            ((y3_ref, wdw2_s3, wpw2_s3, 3, o_s3),
             (y5_ref, wdw2_s5, wpw2_s5, 5, o_s5))):
        y = y_ref[...].reshape(BC, HW).astype(jnp.float32)
        a = jnp.maximum(y * bn[:, 2 * j:2 * j + 1] + bn[:, 2 * j + 1:2 * j + 2],
                        0.0)
        _fill(scr, a, 0.0, BC=BC, pad=pad, HW=HW)
        dw = _dw_conv(_make_slices(scr, pad=pad, W=W, HW=HW), wdw, K, 1,
                      col_mask)
        for b in range(B):
            out = jnp.dot(wpw[...], dw[b * C:(b + 1) * C],
                          preferred_element_type=jnp.float32)
            stats_ref[b, 2 * j] = jnp.sum(out, axis=1, keepdims=True)
            stats_ref[b, 2 * j + 1] = jnp.sum(out * out, axis=1, keepdims=True)
            o_ref[b] = out.astype(jnp.bfloat16)


def _pass2(y3, y5, rc, weights, *, B, N, C, H, W, pad):
    HW = H * W
    img_spec = pl.BlockSpec((B, C, HW), lambda n: (n, 0, 0))
    return pl.pallas_call(
        functools.partial(_p2_kernel, B=B, C=C, H=H, W=W, pad=pad),
        grid=(N // B,),
        in_specs=[_full_spec(rc.shape), img_spec, img_spec]
        + [_full_spec(w.shape) for w in weights],
        out_specs=[img_spec, img_spec,
                   pl.BlockSpec((B, 4, C, 1), lambda n: (n, 0, 0, 0))],
        out_shape=[jax.ShapeDtypeStruct((N, C, HW), jnp.bfloat16)] * 2
        + [jax.ShapeDtypeStruct((N, 4, C, 1), jnp.float32)],
        scratch_shapes=[pltpu.VMEM((B * C, HW + 2 * pad), jnp.float32)],
        compiler_params=_PARAMS_1D,
    )(rc, y3, y5, *weights)


def _p3_kernel(rc_ref, inv_ref, x_ref, s3_ref, s5_ref,
               wdw_d3, wpw_d3, wdw_d5, wpw_d5, sb_ref,
               o_ref, relu_scr, raw_scr, max_scr,
               *, B, C, H, W, pad):
    HW = H * W
    BC = B * C
    col_mask = _make_col_mask(rc_ref, W=W)
    geo = dict(pad=pad, W=W, HW=HW)

    x = x_ref[...].reshape(BC, HW)
    _fill(relu_scr, jnp.maximum(x, 0.0), 0.0, BC=BC, pad=pad, HW=HW)
    _fill(raw_scr, x, 0.0, BC=BC, pad=pad, HW=HW)
    _fill(max_scr, x, _NEG, BC=BC, pad=pad, HW=HW)

    sb = sb_ref[...]
    acc = x * sb[:, 4:5] + sb[:, 5:6]
    acc = acc + s3_ref[...].reshape(BC, HW).astype(jnp.float32) * sb[:, 0:1]
    acc = acc + s5_ref[...].reshape(BC, HW).astype(jnp.float32) * sb[:, 1:2]
    acc = acc + _max_pool(_make_slices(max_scr, **geo), col_mask) * sb[:, 2:3]
    acc = acc + _avg_pool(_make_slices(raw_scr, **geo), inv_ref[...],
                          col_mask) * sb[:, 3:4]
    get_r = _make_slices(relu_scr, **geo)
    dw3 = _dw_conv(get_r, wdw_d3, 3, 2, col_mask)
    dw5 = _dw_conv(get_r, wdw_d5, 5, 2, col_mask)
    for b in range(B):
        sl = slice(b * C, (b + 1) * C)
        o_ref[b] = (acc[sl]
                    + jnp.dot(wpw_d3[...], dw3[sl],
                              preferred_element_type=jnp.float32)
                    + jnp.dot(wpw_d5[...], dw5[sl],
                              preferred_element_type=jnp.float32))


def _pass3(xf, s3, s5, rc, inv_cnt, weights, *, B, N, C, H, W, pad):
    HW = H * W
    img_spec = pl.BlockSpec((B, C, HW), lambda n: (n, 0, 0))
    return pl.pallas_call(
        functools.partial(_p3_kernel, B=B, C=C, H=H, W=W, pad=pad),
        grid=(N // B,),
        in_specs=[_full_spec(rc.shape), _full_spec(inv_cnt.shape),
                  img_spec, img_spec, img_spec]
        + [_full_spec(w.shape) for w in weights],
        out_specs=img_spec,
        out_shape=jax.ShapeDtypeStruct((N, C, HW), jnp.float32),
        scratch_shapes=[pltpu.VMEM((B * C, HW + 2 * pad), jnp.float32)] * 3,
        compiler_params=_PARAMS_1D,
    )(rc, inv_cnt, xf, s3, s5, *weights)


def kernel(x, sep3_dw1, sep3_pw1, sep3_dw2, sep3_pw2,
           sep5_dw1, sep5_pw1, sep5_dw2, sep5_pw2,
           dil3_dw, dil3_pw, dil5_dw, dil5_pw, alphas):
    N, C, H, W = x.shape
    HW = H * W
    f32 = jnp.float32
    x = x.astype(f32)
    xf = x.reshape(N, C, HW)
    alphas = jnp.asarray(alphas, f32)

    B = 4 if N % 4 == 0 else 1
    pad = ((4 * W + 4 + 127) // 128) * 128

    idx = jnp.arange(HW, dtype=jnp.int32)
    rows = idx // W
    cols = idx % W
    rc = jnp.stack([rows, cols]).astype(jnp.int32)
    cnt = ((jnp.minimum(rows, 1) + jnp.minimum(H - 1 - rows, 1) + 1)
           * (jnp.minimum(cols, 1) + jnp.minimum(W - 1 - cols, 1) + 1))
    inv_cnt = (1.0 / cnt.astype(f32)).reshape(1, HW)

    dwW = lambda a: jnp.tile(a.reshape(C, -1).astype(f32), (B, 1))
    pwW = lambda a: a[:, :, 0, 0].astype(f32)

    w1 = (dwW(sep3_dw1), pwW(sep3_pw1), dwW(sep5_dw1), pwW(sep5_pw1),
          dwW(dil3_dw), pwW(dil3_pw), dwW(dil5_dw), pwW(dil5_pw))
    y_s3a, y_s5a, stats1 = _pass1(xf, rc, inv_cnt, w1,
                                  B=B, N=N, C=C, H=H, W=W, pad=pad)

    total = jnp.float32(N * HW)
    st1 = jnp.sum(stats1[..., 0], axis=0)

    def finalize(st, j):
        s, ss = st[2 * j], st[2 * j + 1]
        m = s / total
        v = jnp.maximum(ss / total - m * m, 0.0)
        sc = lax.rsqrt(v + _EPS)
        return sc, -m * sc

    sc_s3a, bi_s3a = finalize(st1, 0)
    sc_s5a, bi_s5a = finalize(st1, 1)
    bn_mid = jnp.tile(jnp.stack([sc_s3a, bi_s3a, sc_s5a, bi_s5a], axis=1),
                      (B, 1))

    w2 = (dwW(sep3_dw2), pwW(sep3_pw2), dwW(sep5_dw2), pwW(sep5_pw2), bn_mid)
    y_s3, y_s5, stats2 = _pass2(y_s3a, y_s5a, rc, w2,
                                B=B, N=N, C=C, H=H, W=W, pad=pad)
    st2 = jnp.sum(stats2[..., 0], axis=0)

    sc_d3, bi_d3 = finalize(st1, 2)
    sc_d5, bi_d5 = finalize(st1, 3)
    sc_mx, bi_mx = finalize(st1, 4)
    sc_av, bi_av = finalize(st1, 5)
    sc_s3, bi_s3 = finalize(st2, 0)
    sc_s5, bi_s5 = finalize(st2, 1)

    total_bias = (alphas[0] * bi_mx + alphas[1] * bi_av
                  + alphas[3] * bi_s3 + alphas[4] * bi_s5
                  + alphas[5] * bi_d3 + alphas[6] * bi_d5)
    zeros = jnp.zeros((C,), f32)
    sb = jnp.tile(jnp.stack([alphas[3] * sc_s3, alphas[4] * sc_s5,
                             alphas[0] * sc_mx, alphas[1] * sc_av,
                             jnp.full((C,), alphas[2], f32), total_bias,
                             zeros, zeros], axis=1), (B, 1))

    wpw_d3f = (alphas[5] * sc_d3)[:, None] * pwW(dil3_pw)
    wpw_d5f = (alphas[6] * sc_d5)[:, None] * pwW(dil5_pw)
    w3 = (dwW(dil3_dw), wpw_d3f, dwW(dil5_dw), wpw_d5f, sb)

    out = _pass3(xf, y_s3, y_s5, rc, inv_cnt, w3,
                 B=B, N=N, C=C, H=H, W=W, pad=pad)
    return out.reshape(N, C, H, W)
